# baseline (speedup 1.0000x reference)
import functools

import jax
import jax.numpy as jnp
from jax import lax
from jax.experimental import pallas as pl
from jax.experimental.pallas import tpu as pltpu
from jax.experimental.pallas import tpu_sc as plsc

N_CLASSES = 1000
BATCH = 16384

NC = 2
NS = 16
L = 16
NW = NC * NS
ROWS_PER_W = BATCH // NW
R_CHUNK = 32
N_CHUNKS = ROWS_PER_W // R_CHUNK

_mesh = plsc.VectorSubcoreMesh(core_axis_name="c", subcore_axis_name="s")


@functools.partial(
    pl.kernel,
    out_type=jax.ShapeDtypeStruct((BATCH, N_CLASSES), jnp.float32),
    mesh=_mesh,
    scratch_types=[
        pltpu.VMEM((ROWS_PER_W,), jnp.int32),
        pltpu.VMEM((R_CHUNK, N_CLASSES), jnp.float32),
        pltpu.VMEM((R_CHUNK, N_CLASSES), jnp.float32),
        pltpu.SemaphoreType.DMA,
        pltpu.SemaphoreType.DMA,
    ],
    compiler_params=pltpu.CompilerParams(
        needs_layout_passes=False, use_tc_tiling_on_sc=True),
)
def _one_hot_sc(idx_hbm, out_hbm, idx_v, buf0, buf1, sem0, sem1):
    wid = lax.axis_index("s") * NC + lax.axis_index("c")
    row0 = wid * ROWS_PER_W

    pltpu.sync_copy(idx_hbm.at[pl.ds(row0, ROWS_PER_W)], idx_v)

    zeros16 = jnp.zeros((L,), jnp.float32)
    ones16 = jnp.ones((L,), jnp.float32)
    lane = lax.iota(jnp.int32, L)

    def _zero(r, carry):
        for j in range(62):
            buf0[r, pl.ds(j * L, L)] = zeros16
            buf1[r, pl.ds(j * L, L)] = zeros16
        buf0[r, pl.ds(N_CLASSES - L, L)] = zeros16
        buf1[r, pl.ds(N_CLASSES - L, L)] = zeros16
        return carry

    lax.fori_loop(0, R_CHUNK, _zero, 0)

    def _flip(buf, chunk, vals):
        for g in range(R_CHUNK // L):
            idxv = idx_v[pl.ds(chunk * R_CHUNK + g * L, L)]
            plsc.store_scatter(buf, (lane + (g * L), idxv), vals)

    bufs = (buf0, buf1)
    sems = (sem0, sem1)
    dmas = [None, None]
    for c in range(N_CHUNKS):
        b = c % 2
        buf = bufs[b]
        if dmas[b] is not None:
            dmas[b].wait()
            _flip(buf, c - 2, zeros16)
        _flip(buf, c, ones16)
        dmas[b] = pltpu.async_copy(
            buf, out_hbm.at[pl.ds(row0 + c * R_CHUNK, R_CHUNK)], sems[b])
    dmas[0].wait()
    dmas[1].wait()


def kernel(inputs):
    return _one_hot_sc(inputs.astype(jnp.int32))


# speedup vs baseline: 1.8200x; 1.8200x over previous
import functools

import jax
import jax.numpy as jnp
from jax import lax
from jax.experimental import pallas as pl
from jax.experimental.pallas import tpu as pltpu
from jax.experimental.pallas import tpu_sc as plsc

N_CLASSES = 1000
BATCH = 16384

NC = 2
NS = 16
L = 16
NW = NC * NS
COLS_PER_W = BATCH // NW
C_BLK = 128
N_BLKS = COLS_PER_W // C_BLK

_mesh = plsc.VectorSubcoreMesh(core_axis_name="c", subcore_axis_name="s")


@functools.partial(
    pl.kernel,
    out_type=jax.ShapeDtypeStruct((N_CLASSES, BATCH), jnp.float32),
    mesh=_mesh,
    scratch_types=[
        pltpu.VMEM((COLS_PER_W,), jnp.int32),
        pltpu.VMEM((N_CLASSES, C_BLK), jnp.float32),
    ],
    compiler_params=pltpu.CompilerParams(needs_layout_passes=False),
)
def _one_hot_t_sc(idx_hbm, z_hbm, out_hbm, idx_v, buf):
    wid = lax.axis_index("s") * NC + lax.axis_index("c")
    col0 = wid * COLS_PER_W

    pltpu.sync_copy(idx_hbm.at[pl.ds(col0, COLS_PER_W)], idx_v)
    pltpu.sync_copy(z_hbm, buf)

    zeros16 = jnp.zeros((L,), jnp.float32)
    ones16 = jnp.ones((L,), jnp.float32)
    lane = lax.iota(jnp.int32, L)

    def _flip(blk, vals):
        for g in range(C_BLK // L):
            idxv = idx_v[pl.ds(blk * C_BLK + g * L, L)]
            plsc.store_scatter(buf, (idxv, lane + (g * L)), vals)

    for c in range(N_BLKS):
        _flip(c, ones16)
        pltpu.sync_copy(buf, out_hbm.at[:, pl.ds(col0 + c * C_BLK, C_BLK)])
        if c + 1 < N_BLKS:
            _flip(c, zeros16)


def kernel(inputs):
    idx = inputs.astype(jnp.int32)
    zblk = jnp.zeros((N_CLASSES, C_BLK), jnp.float32)
    out_t = _one_hot_t_sc(idx, zblk)
    return out_t.T
